# baseline (speedup 1.0000x reference)
import functools

import jax
import jax.numpy as jnp
from jax import lax
from jax.experimental import pallas as pl
from jax.experimental.pallas import tpu as pltpu
from jax.experimental.pallas import tpu_sc as plsc

NN = 10000
EE = 320000
NPAD = 10240
EPAD = 327680
B = 128
NC = 2
NS = 16
ROWS_PER_TILE = NPAD // NS
PAD_SRC = NN
PAD_DST = NN + 16
BP = 64
IDXBUF = 32
NBUF = 4
PROP_CHUNKS_FS = EPAD // (NS * BP)
PROP_CHUNKS_ES = EPAD // (NC * NS * BP)
DEG_CHUNKS = EPAD // (NC * NS * B)

_MESH = plsc.VectorSubcoreMesh(core_axis_name="c", subcore_axis_name="s")
_F32 = jnp.float32


def _fill_rows(buf, nrows, ncols, value):
    vec = jnp.full((16,), value, _F32)

    def body(i, carry):
        for j in range(ncols // 16):
            buf[i, pl.ds(j * 16, 16)] = vec
        return carry

    lax.fori_loop(0, nrows, body, 0)


def _zero_acc_and_sync(r0, acc, sid, nb):
    _fill_rows(r0, nb, 128, 0.0)
    for r in range(ROWS_PER_TILE // nb):
        pltpu.sync_copy(r0, acc.at[pl.ds(sid * ROWS_PER_TILE + r * nb, nb)])


def _writeback(acc, out_hbm, sid, cid, bufs, sems, nb):
    del bufs
    row = sid * ROWS_PER_TILE
    pltpu.async_copy(acc.at[pl.ds(row, ROWS_PER_TILE)],
                     out_hbm.at[pl.ds(cid * NPAD + row, ROWS_PER_TILE)], sems[0])
    pltpu.make_async_copy(acc.at[pl.ds(row, ROWS_PER_TILE)],
                          out_hbm.at[pl.ds(cid * NPAD + row, ROWS_PER_TILE)],
                          sems[0]).wait()


def _make_prop(nchunk, edge_split):
    nstage = nchunk // IDXBUF
    ngroup = IDXBUF // NBUF
    assert nchunk == nstage * IDXBUF and IDXBUF == ngroup * NBUF

    @functools.partial(
        pl.kernel,
        out_type=jax.ShapeDtypeStruct((NC * NPAD, 128), _F32),
        mesh=_MESH,
        scratch_types=[
            pltpu.VMEM((IDXBUF, BP), jnp.int32),
            pltpu.VMEM((IDXBUF, BP), jnp.int32),
            [pltpu.VMEM((BP, 128), _F32)] * NBUF,
            [pltpu.SemaphoreType.DMA] * NBUF,
            [pltpu.SemaphoreType.DMA] * NBUF,
            pltpu.VMEM_SHARED((NPAD, 128), _F32),
        ],
    )
    def prop(y_hbm, srcr_hbm, dstr_hbm, out_hbm,
             sidx, didx, bufs, gsems, ssems, acc):
        cid = lax.axis_index("c")
        sid = lax.axis_index("s")
        if edge_split:
            srow = cid * (EPAD // BP) + (cid * NS + sid) * nchunk
            drow = (cid * NS + sid) * nchunk
        else:
            srow = cid * (EPAD // BP) + sid * nchunk
            drow = sid * nchunk
        _zero_acc_and_sync(bufs[0], acc, sid, BP)
        plsc.subcore_barrier()

        def g_start(b, k):
            pltpu.async_copy(y_hbm.at[sidx.at[k]], bufs[b], gsems[b])

        def g_wait(b):
            pltpu.make_async_copy(y_hbm.at[sidx.at[0]], bufs[b], gsems[b]).wait()

        def s_start(b, k):
            pltpu.async_copy(bufs[b], acc.at[didx.at[k]], ssems[b], add=True)

        def s_wait(b):
            pltpu.make_async_copy(bufs[b], acc.at[didx.at[0]], ssems[b]).wait()

        def body(j, carry):
            for b in range(NBUF):
                k = j * NBUF + b
                g_wait(b)
                s_start(b, k)
                s_wait(b)
                g_start(b, k + NBUF)
            return carry

        for s in range(nstage):
            pltpu.sync_copy(srcr_hbm.at[pl.ds(srow + s * IDXBUF, IDXBUF)], sidx)
            pltpu.sync_copy(dstr_hbm.at[pl.ds(drow + s * IDXBUF, IDXBUF)], didx)
            for b in range(NBUF):
                g_start(b, b)
            lax.fori_loop(0, ngroup - 1, body, 0)
            for b in range(NBUF):
                k = (ngroup - 1) * NBUF + b
                g_wait(b)
                s_start(b, k)
                s_wait(b)
        plsc.subcore_barrier()
        _writeback(acc, out_hbm, sid, cid, (bufs[0], bufs[1]),
                   (gsems[0], gsems[1]), BP)

    return prop


_prop_feature_split = _make_prop(PROP_CHUNKS_FS, edge_split=False)
_prop_edge_split = _make_prop(PROP_CHUNKS_ES, edge_split=True)


@functools.partial(
    pl.kernel,
    out_type=jax.ShapeDtypeStruct((NC * NPAD, 128), _F32),
    mesh=_MESH,
    scratch_types=[
        pltpu.VMEM((DEG_CHUNKS, B), jnp.int32),
        pltpu.VMEM((B, 128), _F32),
        pltpu.VMEM((B, 128), _F32),
        pltpu.SemaphoreType.DMA,
        pltpu.SemaphoreType.DMA,
        pltpu.VMEM_SHARED((NPAD, 128), _F32),
    ],
)
def _deg_sc(dstr_hbm, out_hbm, didx, r0, r1, ss0, ss1, acc):
    cid = lax.axis_index("c")
    sid = lax.axis_index("s")
    drow = (cid * NS + sid) * DEG_CHUNKS
    pltpu.sync_copy(dstr_hbm.at[pl.ds(drow, DEG_CHUNKS)], didx)
    _zero_acc_and_sync(r0, acc, sid, B)
    _fill_rows(r1, B, 128, 1.0)
    plsc.subcore_barrier()

    def s_start(sem, k):
        pltpu.async_copy(r1, acc.at[didx.at[k]], sem, add=True)

    def s_wait(sem):
        pltpu.make_async_copy(r1, acc.at[didx.at[0]], sem).wait()

    s_start(ss0, 0)
    s_start(ss1, 1)

    def body(j, carry):
        s_wait(ss0)
        s_start(ss0, 2 * j + 2)
        s_wait(ss1)
        s_start(ss1, 2 * j + 3)
        return carry

    lax.fori_loop(0, DEG_CHUNKS // 2 - 1, body, 0)
    s_wait(ss0)
    s_wait(ss1)
    plsc.subcore_barrier()
    _writeback(acc, out_hbm, sid, cid, (r0, r1), (ss0, ss1), B)


_R = 1024
_GRID = NPAD // _R


def _row_mask(i, rows):
    idx = i * rows + lax.broadcasted_iota(jnp.int32, (rows, 1), 0)
    return idx < NN


def _mm1_body(d_ref, x_ref, w_ref, dv_out, y_ref):
    i = pl.program_id(0)
    d = d_ref[...]
    deg = d[0, :, 0:1] + d[1, :, 0:1] + 1.0
    dv = lax.rsqrt(jnp.maximum(deg, 1e-12))
    dv_out[...] = jnp.broadcast_to(dv, (_R, 128))
    xw = jnp.dot(x_ref[...], w_ref[...], preferred_element_type=_F32,
                 precision=lax.Precision.HIGHEST)
    y = jnp.where(_row_mask(i, _R), dv * xw, 0.0)
    y_ref[...] = jnp.stack([y[:, :128], y[:, 128:]], axis=0)


def _mm1_tc(deg_parts, x, W1):
    return pl.pallas_call(
        _mm1_body,
        grid=(_GRID,),
        in_specs=[
            pl.BlockSpec((NC, _R, 128), lambda i: (0, i, 0)),
            pl.BlockSpec((_R, 128), lambda i: (i, 0)),
            pl.BlockSpec((128, 256), lambda i: (0, 0)),
        ],
        out_specs=[
            pl.BlockSpec((_R, 128), lambda i: (i, 0)),
            pl.BlockSpec((NC, _R, 128), lambda i: (0, i, 0)),
        ],
        out_shape=[
            jax.ShapeDtypeStruct((NPAD, 128), _F32),
            jax.ShapeDtypeStruct((NC, NPAD, 128), _F32),
        ],
    )(deg_parts, x, W1)


def _mm_mid_body(fout, a_ref, y_ref, dv_ref, b_ref, w_ref, o_ref):
    i = pl.program_id(0)
    s = a_ref[...] + y_ref[...]
    s2 = jnp.concatenate([s[0], s[1]], axis=1)
    dv = dv_ref[...][:, 0:1]
    h = jnp.maximum(dv * s2 + b_ref[...], 0.0)
    xw = jnp.dot(h, w_ref[...], preferred_element_type=_F32,
                 precision=lax.Precision.HIGHEST)
    y = jnp.where(_row_mask(i, _R), dv * xw, 0.0)
    if fout == 256:
        o_ref[...] = jnp.stack([y[:, :128], y[:, 128:]], axis=0)
    else:
        o_ref[...] = jnp.stack([y, y], axis=0)


def _mm_mid_tc(agg, y_prev, dinv, b, W, fout):
    out_shape = jax.ShapeDtypeStruct((NC, NPAD, 128), _F32)
    out_spec = pl.BlockSpec((NC, _R, 128), lambda i: (0, i, 0))
    return pl.pallas_call(
        functools.partial(_mm_mid_body, fout),
        grid=(_GRID,),
        in_specs=[
            pl.BlockSpec((NC, _R, 128), lambda i: (0, i, 0)),
            pl.BlockSpec((NC, _R, 128), lambda i: (0, i, 0)),
            pl.BlockSpec((_R, 128), lambda i: (i, 0)),
            pl.BlockSpec((1, 256), lambda i: (0, 0)),
            pl.BlockSpec((256, fout), lambda i: (0, 0)),
        ],
        out_specs=out_spec,
        out_shape=out_shape,
    )(agg, y_prev, dinv, b, W)


def _final_body(a_ref, y_ref, dv_ref, b_ref, o_ref):
    i = pl.program_id(0)
    a = a_ref[...]
    h = dv_ref[...][:, 0:1] * (a[0] + a[1] + y_ref[...][0]) + b_ref[...]
    h = jnp.where(_row_mask(i, _R), h, jnp.inf)
    m = jnp.min(h, axis=0, keepdims=True)

    @pl.when(i == 0)
    def _():
        o_ref[...] = m

    @pl.when(i > 0)
    def _():
        o_ref[...] = jnp.minimum(o_ref[...], m)


def _final_tc(agg_parts, y3, dinv, b3):
    return pl.pallas_call(
        _final_body,
        grid=(_GRID,),
        in_specs=[
            pl.BlockSpec((NC, _R, 128), lambda i: (0, i, 0)),
            pl.BlockSpec((NC, _R, 128), lambda i: (0, i, 0)),
            pl.BlockSpec((_R, 128), lambda i: (i, 0)),
            pl.BlockSpec((1, 128), lambda i: (0, 0)),
        ],
        out_specs=pl.BlockSpec((1, 128), lambda i: (0, 0)),
        out_shape=jax.ShapeDtypeStruct((1, 128), _F32),
    )(agg_parts, y3, dinv, b3)


def kernel(x, edge_index, W1, b1, W2, b2, W3, b3):
    src = edge_index[0]
    dst = edge_index[1]
    npad_e = EPAD - EE
    src_p = jnp.concatenate([src, jnp.full((npad_e,), PAD_SRC, jnp.int32)])
    dst_p = jnp.concatenate([dst, jnp.full((npad_e,), PAD_DST, jnp.int32)])
    src2 = jnp.concatenate([src_p, src_p + NPAD])
    dst_2d = dst_p.reshape(EPAD // B, B)
    src_2dp = src_p.reshape(EPAD // BP, BP)
    dst_2dp = dst_p.reshape(EPAD // BP, BP)
    src2_2dp = src2.reshape(NC * EPAD // BP, BP)
    xp = jnp.pad(x, ((0, NPAD - NN), (0, 0)))
    b1r = b1.reshape(1, 256)
    b2r = b2.reshape(1, 256)
    b3r = b3.reshape(1, 128)

    deg_parts = _deg_sc(dst_2d).reshape(NC, NPAD, 128)
    dinv, y1 = _mm1_tc(deg_parts, xp, W1)
    agg1 = _prop_feature_split(y1.reshape(NC * NPAD, 128), src2_2dp, dst_2dp)
    y2 = _mm_mid_tc(agg1.reshape(NC, NPAD, 128), y1, dinv, b1r, W2, 256)
    agg2 = _prop_feature_split(y2.reshape(NC * NPAD, 128), src2_2dp, dst_2dp)
    y3 = _mm_mid_tc(agg2.reshape(NC, NPAD, 128), y2, dinv, b2r, W3, 128)
    agg3 = _prop_edge_split(y3.reshape(NC * NPAD, 128), src2_2dp, dst_2dp)
    out = _final_tc(agg3.reshape(NC, NPAD, 128), y3, dinv, b3r)
    return out.reshape(128)


# speedup vs baseline: 1.0031x; 1.0031x over previous
import functools

import jax
import jax.numpy as jnp
from jax import lax
from jax.experimental import pallas as pl
from jax.experimental.pallas import tpu as pltpu
from jax.experimental.pallas import tpu_sc as plsc

NN = 10000
EE = 320000
NPAD = 10240
EPAD = 327680
B = 128
NC = 2
NS = 16
ROWS_PER_TILE = NPAD // NS
PAD_SRC = NN
PAD_DST = NN + 16
BP = 64
IDXBUF = 32
NBUF = 4
PROP_CHUNKS_FS = EPAD // (NS * BP)
PROP_CHUNKS_ES = EPAD // (NC * NS * BP)
DEG_CHUNKS = EPAD // (NC * NS * B)

_MESH = plsc.VectorSubcoreMesh(core_axis_name="c", subcore_axis_name="s")
_F32 = jnp.float32


def _fill_rows(buf, nrows, ncols, value):
    vec = jnp.full((16,), value, _F32)

    def body(i, carry):
        for j in range(ncols // 16):
            buf[i, pl.ds(j * 16, 16)] = vec
        return carry

    lax.fori_loop(0, nrows, body, 0)


def _zero_acc_and_sync(r0, acc, sid, nb):
    _fill_rows(r0, nb, 128, 0.0)
    for r in range(ROWS_PER_TILE // nb):
        pltpu.sync_copy(r0, acc.at[pl.ds(sid * ROWS_PER_TILE + r * nb, nb)])


def _writeback(acc, out_hbm, sid, cid, bufs, sems, nb):
    nch = ROWS_PER_TILE // nb
    for r in range(nch):
        row = sid * ROWS_PER_TILE + r * nb
        rb, sem = bufs[r % 2], sems[r % 2]
        if r >= 2:
            prow = cid * NPAD + sid * ROWS_PER_TILE + (r - 2) * nb
            pltpu.make_async_copy(rb, out_hbm.at[pl.ds(prow, nb)], sem).wait()
        pltpu.sync_copy(acc.at[pl.ds(row, nb)], rb)
        pltpu.async_copy(rb, out_hbm.at[pl.ds(cid * NPAD + row, nb)], sem)
    for r in range(max(0, nch - 2), nch):
        row = cid * NPAD + sid * ROWS_PER_TILE + r * nb
        pltpu.make_async_copy(bufs[r % 2], out_hbm.at[pl.ds(row, nb)], sems[r % 2]).wait()


def _make_prop(nchunk, edge_split):
    nstage = nchunk // IDXBUF
    ngroup = IDXBUF // NBUF
    assert nchunk == nstage * IDXBUF and IDXBUF == ngroup * NBUF

    @functools.partial(
        pl.kernel,
        out_type=jax.ShapeDtypeStruct((NC * NPAD, 128), _F32),
        mesh=_MESH,
        scratch_types=[
            pltpu.VMEM((IDXBUF, BP), jnp.int32),
            pltpu.VMEM((IDXBUF, BP), jnp.int32),
            [pltpu.VMEM((BP, 128), _F32)] * NBUF,
            [pltpu.SemaphoreType.DMA] * NBUF,
            [pltpu.SemaphoreType.DMA] * NBUF,
            pltpu.VMEM_SHARED((NPAD, 128), _F32),
        ],
    )
    def prop(y_hbm, srcr_hbm, dstr_hbm, out_hbm,
             sidx, didx, bufs, gsems, ssems, acc):
        cid = lax.axis_index("c")
        sid = lax.axis_index("s")
        if edge_split:
            srow = cid * (EPAD // BP) + (cid * NS + sid) * nchunk
            drow = (cid * NS + sid) * nchunk
        else:
            srow = cid * (EPAD // BP) + sid * nchunk
            drow = sid * nchunk
        _zero_acc_and_sync(bufs[0], acc, sid, BP)
        plsc.subcore_barrier()

        def g_start(b, k):
            pltpu.async_copy(y_hbm.at[sidx.at[k]], bufs[b], gsems[b])

        def g_wait(b):
            pltpu.make_async_copy(y_hbm.at[sidx.at[0]], bufs[b], gsems[b]).wait()

        def s_start(b, k):
            pltpu.async_copy(bufs[b], acc.at[didx.at[k]], ssems[b], add=True)

        def s_wait(b):
            pltpu.make_async_copy(bufs[b], acc.at[didx.at[0]], ssems[b]).wait()

        def body(j, carry):
            for b in range(NBUF):
                k = j * NBUF + b
                g_wait(b)
                s_start(b, k)
                s_wait(b)
                g_start(b, k + NBUF)
            return carry

        for s in range(nstage):
            pltpu.sync_copy(srcr_hbm.at[pl.ds(srow + s * IDXBUF, IDXBUF)], sidx)
            pltpu.sync_copy(dstr_hbm.at[pl.ds(drow + s * IDXBUF, IDXBUF)], didx)
            for b in range(NBUF):
                g_start(b, b)
            lax.fori_loop(0, ngroup - 1, body, 0)
            for b in range(NBUF):
                k = (ngroup - 1) * NBUF + b
                g_wait(b)
                s_start(b, k)
                s_wait(b)
        plsc.subcore_barrier()
        _writeback(acc, out_hbm, sid, cid, (bufs[0], bufs[1]),
                   (gsems[0], gsems[1]), BP)

    return prop


_prop_feature_split = _make_prop(PROP_CHUNKS_FS, edge_split=False)
_prop_edge_split = _make_prop(PROP_CHUNKS_ES, edge_split=True)


@functools.partial(
    pl.kernel,
    out_type=jax.ShapeDtypeStruct((NC * NPAD, 128), _F32),
    mesh=_MESH,
    scratch_types=[
        pltpu.VMEM((DEG_CHUNKS, B), jnp.int32),
        pltpu.VMEM((B, 128), _F32),
        pltpu.VMEM((B, 128), _F32),
        pltpu.SemaphoreType.DMA,
        pltpu.SemaphoreType.DMA,
        pltpu.VMEM_SHARED((NPAD, 128), _F32),
    ],
)
def _deg_sc(dstr_hbm, out_hbm, didx, r0, r1, ss0, ss1, acc):
    cid = lax.axis_index("c")
    sid = lax.axis_index("s")
    drow = (cid * NS + sid) * DEG_CHUNKS
    pltpu.sync_copy(dstr_hbm.at[pl.ds(drow, DEG_CHUNKS)], didx)
    _zero_acc_and_sync(r0, acc, sid, B)
    _fill_rows(r1, B, 128, 1.0)
    plsc.subcore_barrier()

    def s_start(sem, k):
        pltpu.async_copy(r1, acc.at[didx.at[k]], sem, add=True)

    def s_wait(sem):
        pltpu.make_async_copy(r1, acc.at[didx.at[0]], sem).wait()

    s_start(ss0, 0)
    s_start(ss1, 1)

    def body(j, carry):
        s_wait(ss0)
        s_start(ss0, 2 * j + 2)
        s_wait(ss1)
        s_start(ss1, 2 * j + 3)
        return carry

    lax.fori_loop(0, DEG_CHUNKS // 2 - 1, body, 0)
    s_wait(ss0)
    s_wait(ss1)
    plsc.subcore_barrier()
    _writeback(acc, out_hbm, sid, cid, (r0, r1), (ss0, ss1), B)


_R = 1024
_GRID = NPAD // _R


def _row_mask(i, rows):
    idx = i * rows + lax.broadcasted_iota(jnp.int32, (rows, 1), 0)
    return idx < NN


def _mm1_body(d_ref, x_ref, w_ref, dv_out, y_ref):
    i = pl.program_id(0)
    d = d_ref[...]
    deg = d[0, :, 0:1] + d[1, :, 0:1] + 1.0
    dv = lax.rsqrt(jnp.maximum(deg, 1e-12))
    dv_out[...] = jnp.broadcast_to(dv, (_R, 128))
    xw = jnp.dot(x_ref[...], w_ref[...], preferred_element_type=_F32,
                 precision=lax.Precision.HIGHEST)
    y = jnp.where(_row_mask(i, _R), dv * xw, 0.0)
    y_ref[...] = jnp.stack([y[:, :128], y[:, 128:]], axis=0)


def _mm1_tc(deg_parts, x, W1):
    return pl.pallas_call(
        _mm1_body,
        grid=(_GRID,),
        in_specs=[
            pl.BlockSpec((NC, _R, 128), lambda i: (0, i, 0)),
            pl.BlockSpec((_R, 128), lambda i: (i, 0)),
            pl.BlockSpec((128, 256), lambda i: (0, 0)),
        ],
        out_specs=[
            pl.BlockSpec((_R, 128), lambda i: (i, 0)),
            pl.BlockSpec((NC, _R, 128), lambda i: (0, i, 0)),
        ],
        out_shape=[
            jax.ShapeDtypeStruct((NPAD, 128), _F32),
            jax.ShapeDtypeStruct((NC, NPAD, 128), _F32),
        ],
    )(deg_parts, x, W1)


def _mm_mid_body(fout, a_ref, y_ref, dv_ref, b_ref, w_ref, o_ref):
    i = pl.program_id(0)
    s = a_ref[...] + y_ref[...]
    s2 = jnp.concatenate([s[0], s[1]], axis=1)
    dv = dv_ref[...][:, 0:1]
    h = jnp.maximum(dv * s2 + b_ref[...], 0.0)
    xw = jnp.dot(h, w_ref[...], preferred_element_type=_F32,
                 precision=lax.Precision.HIGHEST)
    y = jnp.where(_row_mask(i, _R), dv * xw, 0.0)
    if fout == 256:
        o_ref[...] = jnp.stack([y[:, :128], y[:, 128:]], axis=0)
    else:
        o_ref[...] = jnp.stack([y, y], axis=0)


def _mm_mid_tc(agg, y_prev, dinv, b, W, fout):
    out_shape = jax.ShapeDtypeStruct((NC, NPAD, 128), _F32)
    out_spec = pl.BlockSpec((NC, _R, 128), lambda i: (0, i, 0))
    return pl.pallas_call(
        functools.partial(_mm_mid_body, fout),
        grid=(_GRID,),
        in_specs=[
            pl.BlockSpec((NC, _R, 128), lambda i: (0, i, 0)),
            pl.BlockSpec((NC, _R, 128), lambda i: (0, i, 0)),
            pl.BlockSpec((_R, 128), lambda i: (i, 0)),
            pl.BlockSpec((1, 256), lambda i: (0, 0)),
            pl.BlockSpec((256, fout), lambda i: (0, 0)),
        ],
        out_specs=out_spec,
        out_shape=out_shape,
    )(agg, y_prev, dinv, b, W)


def _final_body(a_ref, y_ref, dv_ref, b_ref, o_ref):
    i = pl.program_id(0)
    a = a_ref[...]
    h = dv_ref[...][:, 0:1] * (a[0] + a[1] + y_ref[...][0]) + b_ref[...]
    h = jnp.where(_row_mask(i, _R), h, jnp.inf)
    m = jnp.min(h, axis=0, keepdims=True)

    @pl.when(i == 0)
    def _():
        o_ref[...] = m

    @pl.when(i > 0)
    def _():
        o_ref[...] = jnp.minimum(o_ref[...], m)


def _final_tc(agg_parts, y3, dinv, b3):
    return pl.pallas_call(
        _final_body,
        grid=(_GRID,),
        in_specs=[
            pl.BlockSpec((NC, _R, 128), lambda i: (0, i, 0)),
            pl.BlockSpec((NC, _R, 128), lambda i: (0, i, 0)),
            pl.BlockSpec((_R, 128), lambda i: (i, 0)),
            pl.BlockSpec((1, 128), lambda i: (0, 0)),
        ],
        out_specs=pl.BlockSpec((1, 128), lambda i: (0, 0)),
        out_shape=jax.ShapeDtypeStruct((1, 128), _F32),
    )(agg_parts, y3, dinv, b3)


def kernel(x, edge_index, W1, b1, W2, b2, W3, b3):
    src = edge_index[0]
    dst = edge_index[1]
    npad_e = EPAD - EE
    src_p = jnp.concatenate([src, jnp.full((npad_e,), PAD_SRC, jnp.int32)])
    dst_p = jnp.concatenate([dst, jnp.full((npad_e,), PAD_DST, jnp.int32)])
    src2 = jnp.concatenate([src_p, src_p + NPAD])
    dst_2d = dst_p.reshape(EPAD // B, B)
    src_2dp = src_p.reshape(EPAD // BP, BP)
    dst_2dp = dst_p.reshape(EPAD // BP, BP)
    src2_2dp = src2.reshape(NC * EPAD // BP, BP)
    xp = jnp.pad(x, ((0, NPAD - NN), (0, 0)))
    b1r = b1.reshape(1, 256)
    b2r = b2.reshape(1, 256)
    b3r = b3.reshape(1, 128)

    deg_parts = _deg_sc(dst_2d).reshape(NC, NPAD, 128)
    dinv, y1 = _mm1_tc(deg_parts, xp, W1)
    agg1 = _prop_feature_split(y1.reshape(NC * NPAD, 128), src2_2dp, dst_2dp)
    y2 = _mm_mid_tc(agg1.reshape(NC, NPAD, 128), y1, dinv, b1r, W2, 256)
    agg2 = _prop_feature_split(y2.reshape(NC * NPAD, 128), src2_2dp, dst_2dp)
    y3 = _mm_mid_tc(agg2.reshape(NC, NPAD, 128), y2, dinv, b2r, W3, 128)
    agg3 = _prop_edge_split(y3.reshape(NC * NPAD, 128), src2_2dp, dst_2dp)
    out = _final_tc(agg3.reshape(NC, NPAD, 128), y3, dinv, b3r)
    return out.reshape(128)


# speedup vs baseline: 1.0446x; 1.0414x over previous
import functools

import jax
import jax.numpy as jnp
from jax import lax
from jax.experimental import pallas as pl
from jax.experimental.pallas import tpu as pltpu
from jax.experimental.pallas import tpu_sc as plsc

NN = 10000
EE = 320000
NPAD = 10240
EPAD = 327680
B = 128
NC = 2
NS = 16
ROWS_PER_TILE = NPAD // NS
PAD_SRC = NN
PAD_DST = NN + 16
BP = 64
IDXBUF = 32
NBUF = 4
PROP_CHUNKS_FS = EPAD // (NS * BP)
PROP_CHUNKS_ES = EPAD // (NC * NS * BP)
DEG_CHUNKS = EPAD // (NC * NS * B)

_MESH = plsc.VectorSubcoreMesh(core_axis_name="c", subcore_axis_name="s")
_F32 = jnp.float32


def _fill_rows(buf, nrows, ncols, value):
    vec = jnp.full((16,), value, _F32)

    def body(i, carry):
        for j in range(ncols // 16):
            buf[i, pl.ds(j * 16, 16)] = vec
        return carry

    lax.fori_loop(0, nrows, body, 0)


def _zero_acc_and_sync(r0, acc, sid, nb):
    _fill_rows(r0, nb, 128, 0.0)
    for r in range(ROWS_PER_TILE // nb):
        pltpu.sync_copy(r0, acc.at[pl.ds(sid * ROWS_PER_TILE + r * nb, nb)])


def _writeback(acc, out_hbm, sid, cid, bufs, sems, nb):
    nch = ROWS_PER_TILE // nb
    for r in range(nch):
        row = sid * ROWS_PER_TILE + r * nb
        rb, sem = bufs[r % 2], sems[r % 2]
        if r >= 2:
            prow = cid * NPAD + sid * ROWS_PER_TILE + (r - 2) * nb
            pltpu.make_async_copy(rb, out_hbm.at[pl.ds(prow, nb)], sem).wait()
        pltpu.sync_copy(acc.at[pl.ds(row, nb)], rb)
        pltpu.async_copy(rb, out_hbm.at[pl.ds(cid * NPAD + row, nb)], sem)
    for r in range(max(0, nch - 2), nch):
        row = cid * NPAD + sid * ROWS_PER_TILE + r * nb
        pltpu.make_async_copy(bufs[r % 2], out_hbm.at[pl.ds(row, nb)], sems[r % 2]).wait()


def _make_prop(nchunk, edge_split):
    nstage = nchunk // IDXBUF
    ngroup = IDXBUF // NBUF
    assert nchunk == nstage * IDXBUF and IDXBUF == ngroup * NBUF

    @functools.partial(
        pl.kernel,
        out_type=jax.ShapeDtypeStruct((NC * NPAD, 128), _F32),
        mesh=_MESH,
        scratch_types=[
            pltpu.VMEM((IDXBUF, BP), jnp.int32),
            pltpu.VMEM((IDXBUF, BP), jnp.int32),
            [pltpu.VMEM((BP, 128), _F32)] * NBUF,
            [pltpu.SemaphoreType.DMA] * NBUF,
            [pltpu.SemaphoreType.DMA] * NBUF,
            pltpu.VMEM_SHARED((NPAD, 128), _F32),
        ],
    )
    def prop(y_hbm, srcr_hbm, dstr_hbm, out_hbm,
             sidx, didx, bufs, gsems, ssems, acc):
        cid = lax.axis_index("c")
        sid = lax.axis_index("s")
        if edge_split:
            srow = cid * (EPAD // BP) + (cid * NS + sid) * nchunk
            drow = (cid * NS + sid) * nchunk
        else:
            srow = cid * (EPAD // BP) + sid * nchunk
            drow = sid * nchunk
        _zero_acc_and_sync(bufs[0], acc, sid, BP)
        plsc.subcore_barrier()

        def g_start(b, k):
            pltpu.async_copy(y_hbm.at[sidx.at[k]], bufs[b], gsems[b])

        def g_wait(b):
            pltpu.make_async_copy(y_hbm.at[sidx.at[0]], bufs[b], gsems[b]).wait()

        def s_start(b, k):
            pltpu.async_copy(bufs[b], acc.at[didx.at[k]], ssems[b], add=True)

        def s_wait(b):
            pltpu.make_async_copy(bufs[b], acc.at[didx.at[0]], ssems[b]).wait()

        def body(j, carry):
            for b in range(NBUF):
                k = j * NBUF + b
                g_wait(b)
                s_start(b, k)
                s_wait(b)
                g_start(b, k + NBUF)
            return carry

        for s in range(nstage):
            pltpu.sync_copy(srcr_hbm.at[pl.ds(srow + s * IDXBUF, IDXBUF)], sidx)
            pltpu.sync_copy(dstr_hbm.at[pl.ds(drow + s * IDXBUF, IDXBUF)], didx)
            for b in range(NBUF):
                g_start(b, b)
            lax.fori_loop(0, ngroup - 1, body, 0)
            for b in range(NBUF):
                k = (ngroup - 1) * NBUF + b
                g_wait(b)
                s_start(b, k)
                s_wait(b)
        plsc.subcore_barrier()
        _writeback(acc, out_hbm, sid, cid, (bufs[0], bufs[1]),
                   (gsems[0], gsems[1]), BP)

    return prop


_prop_feature_split = _make_prop(PROP_CHUNKS_FS, edge_split=False)
_prop_edge_split = _make_prop(PROP_CHUNKS_ES, edge_split=True)


@functools.partial(
    pl.kernel,
    out_type=jax.ShapeDtypeStruct((NC * NPAD, 128), _F32),
    mesh=_MESH,
    scratch_types=[
        pltpu.VMEM((DEG_CHUNKS, B), jnp.int32),
        pltpu.VMEM((B, 128), _F32),
        pltpu.VMEM((B, 128), _F32),
        pltpu.SemaphoreType.DMA,
        pltpu.SemaphoreType.DMA,
        pltpu.VMEM_SHARED((NPAD, 128), _F32),
    ],
)
def _deg_sc(dstr_hbm, out_hbm, didx, r0, r1, ss0, ss1, acc):
    cid = lax.axis_index("c")
    sid = lax.axis_index("s")
    drow = (cid * NS + sid) * DEG_CHUNKS
    pltpu.sync_copy(dstr_hbm.at[pl.ds(drow, DEG_CHUNKS)], didx)
    _zero_acc_and_sync(r0, acc, sid, B)
    _fill_rows(r1, B, 128, 1.0)
    plsc.subcore_barrier()

    def s_start(sem, k):
        pltpu.async_copy(r1, acc.at[didx.at[k]], sem, add=True)

    def s_wait(sem):
        pltpu.make_async_copy(r1, acc.at[didx.at[0]], sem).wait()

    s_start(ss0, 0)
    s_start(ss1, 1)

    def body(j, carry):
        s_wait(ss0)
        s_start(ss0, 2 * j + 2)
        s_wait(ss1)
        s_start(ss1, 2 * j + 3)
        return carry

    lax.fori_loop(0, DEG_CHUNKS // 2 - 1, body, 0)
    s_wait(ss0)
    s_wait(ss1)
    plsc.subcore_barrier()
    _writeback(acc, out_hbm, sid, cid, (r0, r1), (ss0, ss1), B)


_R = 1024
_GRID = NPAD // _R


def _row_mask(i, rows):
    idx = i * rows + lax.broadcasted_iota(jnp.int32, (rows, 1), 0)
    return idx < NN


def _dinv_body(d_ref, o_ref):
    d = d_ref[...]
    deg = d[0, :, 0:1] + d[1, :, 0:1] + 1.0
    dinv = lax.rsqrt(jnp.maximum(deg, 1e-12))
    o_ref[...] = jnp.broadcast_to(dinv, (_R, 128))


def _dinv_tc(d):
    return pl.pallas_call(
        _dinv_body,
        grid=(_GRID,),
        in_specs=[pl.BlockSpec((NC, _R, 128), lambda i: (0, i, 0))],
        out_specs=pl.BlockSpec((_R, 128), lambda i: (i, 0)),
        out_shape=jax.ShapeDtypeStruct((NPAD, 128), _F32),
    )(d)


def _mm1_body(x_ref, w_ref, dv_ref, o_ref):
    i = pl.program_id(0)
    xw = jnp.dot(x_ref[...], w_ref[...], preferred_element_type=_F32,
                 precision=lax.Precision.HIGHEST)
    dv = dv_ref[...][:, 0:1]
    y = jnp.where(_row_mask(i, _R), dv * xw, 0.0)
    o_ref[...] = jnp.stack([y[:, :128], y[:, 128:]], axis=0)


def _mm1_tc(x, W1, dinv):
    return pl.pallas_call(
        _mm1_body,
        grid=(_GRID,),
        in_specs=[
            pl.BlockSpec((_R, 128), lambda i: (i, 0)),
            pl.BlockSpec((128, 256), lambda i: (0, 0)),
            pl.BlockSpec((_R, 128), lambda i: (i, 0)),
        ],
        out_specs=pl.BlockSpec((NC, _R, 128), lambda i: (0, i, 0)),
        out_shape=jax.ShapeDtypeStruct((NC, NPAD, 128), _F32),
    )(x, W1, dinv)


def _mm_mid_body(fout, a_ref, y_ref, dv_ref, b_ref, w_ref, o_ref):
    i = pl.program_id(0)
    s = a_ref[...] + y_ref[...]
    s2 = jnp.concatenate([s[0], s[1]], axis=1)
    dv = dv_ref[...][:, 0:1]
    h = jnp.maximum(dv * s2 + b_ref[...], 0.0)
    xw = jnp.dot(h, w_ref[...], preferred_element_type=_F32,
                 precision=lax.Precision.HIGHEST)
    y = jnp.where(_row_mask(i, _R), dv * xw, 0.0)
    if fout == 256:
        o_ref[...] = jnp.stack([y[:, :128], y[:, 128:]], axis=0)
    else:
        o_ref[...] = jnp.stack([y, y], axis=0)


def _mm_mid_tc(agg, y_prev, dinv, b, W, fout):
    out_shape = jax.ShapeDtypeStruct((NC, NPAD, 128), _F32)
    out_spec = pl.BlockSpec((NC, _R, 128), lambda i: (0, i, 0))
    return pl.pallas_call(
        functools.partial(_mm_mid_body, fout),
        grid=(_GRID,),
        in_specs=[
            pl.BlockSpec((NC, _R, 128), lambda i: (0, i, 0)),
            pl.BlockSpec((NC, _R, 128), lambda i: (0, i, 0)),
            pl.BlockSpec((_R, 128), lambda i: (i, 0)),
            pl.BlockSpec((1, 256), lambda i: (0, 0)),
            pl.BlockSpec((256, fout), lambda i: (0, 0)),
        ],
        out_specs=out_spec,
        out_shape=out_shape,
    )(agg, y_prev, dinv, b, W)


def _final_body(a_ref, y_ref, dv_ref, b_ref, o_ref):
    i = pl.program_id(0)
    a = a_ref[...]
    h = dv_ref[...][:, 0:1] * (a[0] + a[1] + y_ref[...][0]) + b_ref[...]
    h = jnp.where(_row_mask(i, _R), h, jnp.inf)
    m = jnp.min(h, axis=0, keepdims=True)

    @pl.when(i == 0)
    def _():
        o_ref[...] = m

    @pl.when(i > 0)
    def _():
        o_ref[...] = jnp.minimum(o_ref[...], m)


def _final_tc(agg_parts, y3, dinv, b3):
    return pl.pallas_call(
        _final_body,
        grid=(_GRID,),
        in_specs=[
            pl.BlockSpec((NC, _R, 128), lambda i: (0, i, 0)),
            pl.BlockSpec((NC, _R, 128), lambda i: (0, i, 0)),
            pl.BlockSpec((_R, 128), lambda i: (i, 0)),
            pl.BlockSpec((1, 128), lambda i: (0, 0)),
        ],
        out_specs=pl.BlockSpec((1, 128), lambda i: (0, 0)),
        out_shape=jax.ShapeDtypeStruct((1, 128), _F32),
    )(agg_parts, y3, dinv, b3)


def kernel(x, edge_index, W1, b1, W2, b2, W3, b3):
    src = edge_index[0]
    dst = edge_index[1]
    npad_e = EPAD - EE
    src_p = jnp.concatenate([src, jnp.full((npad_e,), PAD_SRC, jnp.int32)])
    dst_p = jnp.concatenate([dst, jnp.full((npad_e,), PAD_DST, jnp.int32)])
    src2 = jnp.concatenate([src_p, src_p + NPAD])
    dst_2d = dst_p.reshape(EPAD // B, B)
    src_2dp = src_p.reshape(EPAD // BP, BP)
    dst_2dp = dst_p.reshape(EPAD // BP, BP)
    src2_2dp = src2.reshape(NC * EPAD // BP, BP)
    xp = jnp.pad(x, ((0, NPAD - NN), (0, 0)))
    b1r = b1.reshape(1, 256)
    b2r = b2.reshape(1, 256)
    b3r = b3.reshape(1, 128)

    deg_parts = _deg_sc(dst_2d).reshape(NC, NPAD, 128)
    dinv = _dinv_tc(deg_parts)
    y1 = _mm1_tc(xp, W1, dinv)
    agg1 = _prop_feature_split(y1.reshape(NC * NPAD, 128), src2_2dp, dst_2dp)
    y2 = _mm_mid_tc(agg1.reshape(NC, NPAD, 128), y1, dinv, b1r, W2, 256)
    agg2 = _prop_feature_split(y2.reshape(NC * NPAD, 128), src2_2dp, dst_2dp)
    y3 = _mm_mid_tc(agg2.reshape(NC, NPAD, 128), y2, dinv, b2r, W3, 128)
    agg3 = _prop_edge_split(y3.reshape(NC * NPAD, 128), src2_2dp, dst_2dp)
    out = _final_tc(agg3.reshape(NC, NPAD, 128), y3, dinv, b3r)
    return out.reshape(128)


# speedup vs baseline: 1.0558x; 1.0107x over previous
import functools

import jax
import jax.numpy as jnp
from jax import lax
from jax.experimental import pallas as pl
from jax.experimental.pallas import tpu as pltpu
from jax.experimental.pallas import tpu_sc as plsc

NN = 10000
EE = 320000
NPAD = 10240
EPAD = 327680
B = 128
NC = 2
NS = 16
ROWS_PER_TILE = NPAD // NS
PAD_SRC = NN
PAD_DST = NN + 16
BP = 64
IDXBUF = 32
NBUF = 4
PROP_CHUNKS_FS = EPAD // (NS * BP)
PROP_CHUNKS_ES = EPAD // (NC * NS * BP)
DEG_CHUNKS = EPAD // (NC * NS * B)

_MESH = plsc.VectorSubcoreMesh(core_axis_name="c", subcore_axis_name="s")
_F32 = jnp.float32


def _fill_rows(buf, nrows, ncols, value):
    vec = jnp.full((16,), value, _F32)

    def body(i, carry):
        for j in range(ncols // 16):
            buf[i, pl.ds(j * 16, 16)] = vec
        return carry

    lax.fori_loop(0, nrows, body, 0)


def _zero_acc_and_sync(r0, acc, sid, nb):
    _fill_rows(r0, nb, 128, 0.0)
    for r in range(ROWS_PER_TILE // nb):
        pltpu.sync_copy(r0, acc.at[pl.ds(sid * ROWS_PER_TILE + r * nb, nb)])


def _writeback(acc, out_hbm, sid, cid, bufs, sems, nb):
    nch = ROWS_PER_TILE // nb
    for r in range(nch):
        row = sid * ROWS_PER_TILE + r * nb
        rb, sem = bufs[r % 2], sems[r % 2]
        if r >= 2:
            prow = cid * NPAD + sid * ROWS_PER_TILE + (r - 2) * nb
            pltpu.make_async_copy(rb, out_hbm.at[pl.ds(prow, nb)], sem).wait()
        pltpu.sync_copy(acc.at[pl.ds(row, nb)], rb)
        pltpu.async_copy(rb, out_hbm.at[pl.ds(cid * NPAD + row, nb)], sem)
    for r in range(max(0, nch - 2), nch):
        row = cid * NPAD + sid * ROWS_PER_TILE + r * nb
        pltpu.make_async_copy(bufs[r % 2], out_hbm.at[pl.ds(row, nb)], sems[r % 2]).wait()


def _make_prop(nchunk, edge_split):
    nstage = nchunk // IDXBUF
    ngroup = IDXBUF // NBUF
    assert nchunk == nstage * IDXBUF and IDXBUF == ngroup * NBUF

    @functools.partial(
        pl.kernel,
        out_type=jax.ShapeDtypeStruct((NC * NPAD, 128), _F32),
        mesh=_MESH,
        scratch_types=[
            pltpu.VMEM((IDXBUF, BP), jnp.int32),
            pltpu.VMEM((IDXBUF, BP), jnp.int32),
            [pltpu.VMEM((BP, 128), _F32)] * NBUF,
            [pltpu.SemaphoreType.DMA] * NBUF,
            [pltpu.SemaphoreType.DMA] * NBUF,
            pltpu.VMEM_SHARED((NPAD, 128), _F32),
        ],
    )
    def prop(y_hbm, srcr_hbm, dstr_hbm, out_hbm,
             sidx, didx, bufs, gsems, ssems, acc):
        cid = lax.axis_index("c")
        sid = lax.axis_index("s")
        if edge_split:
            srow = cid * (EPAD // BP) + (cid * NS + sid) * nchunk
            drow = (cid * NS + sid) * nchunk
        else:
            srow = cid * (EPAD // BP) + sid * nchunk
            drow = sid * nchunk
        _zero_acc_and_sync(bufs[0], acc, sid, BP)
        plsc.subcore_barrier()

        def g_start(b, k):
            pltpu.async_copy(y_hbm.at[sidx.at[k]], bufs[b], gsems[b])

        def g_wait(b):
            pltpu.make_async_copy(y_hbm.at[sidx.at[0]], bufs[b], gsems[b]).wait()

        def s_start(b, k):
            pltpu.async_copy(bufs[b], acc.at[didx.at[k]], ssems[b], add=True)

        def s_wait(b):
            pltpu.make_async_copy(bufs[b], acc.at[didx.at[0]], ssems[b]).wait()

        def body(j, carry):
            for b in range(NBUF):
                k = j * NBUF + b
                g_wait(b)
                s_start(b, k)
                s_wait(b)
                g_start(b, k + NBUF)
            return carry

        for s in range(nstage):
            pltpu.sync_copy(srcr_hbm.at[pl.ds(srow + s * IDXBUF, IDXBUF)], sidx)
            pltpu.sync_copy(dstr_hbm.at[pl.ds(drow + s * IDXBUF, IDXBUF)], didx)
            for b in range(NBUF):
                g_start(b, b)
            lax.fori_loop(0, ngroup - 1, body, 0)
            for b in range(NBUF):
                k = (ngroup - 1) * NBUF + b
                g_wait(b)
                s_start(b, k)
                s_wait(b)
        plsc.subcore_barrier()
        _writeback(acc, out_hbm, sid, cid, (bufs[0], bufs[1]),
                   (gsems[0], gsems[1]), BP)

    return prop


_prop_feature_split = _make_prop(PROP_CHUNKS_FS, edge_split=False)
_prop_edge_split = _make_prop(PROP_CHUNKS_ES, edge_split=True)


@functools.partial(
    pl.kernel,
    out_type=jax.ShapeDtypeStruct((NC * NPAD, 128), _F32),
    mesh=_MESH,
    scratch_types=[
        pltpu.VMEM((DEG_CHUNKS, B), jnp.int32),
        pltpu.VMEM((B, 128), _F32),
        pltpu.VMEM((B, 128), _F32),
        pltpu.SemaphoreType.DMA,
        pltpu.SemaphoreType.DMA,
        pltpu.VMEM_SHARED((NPAD, 128), _F32),
    ],
)
def _deg_sc(dstr_hbm, out_hbm, didx, r0, r1, ss0, ss1, acc):
    cid = lax.axis_index("c")
    sid = lax.axis_index("s")
    drow = (cid * NS + sid) * DEG_CHUNKS
    pltpu.sync_copy(dstr_hbm.at[pl.ds(drow, DEG_CHUNKS)], didx)
    _zero_acc_and_sync(r0, acc, sid, B)
    _fill_rows(r1, B, 128, 1.0)
    plsc.subcore_barrier()

    def s_start(sem, k):
        pltpu.async_copy(r1, acc.at[didx.at[k]], sem, add=True)

    def s_wait(sem):
        pltpu.make_async_copy(r1, acc.at[didx.at[0]], sem).wait()

    s_start(ss0, 0)
    s_start(ss1, 1)

    def body(j, carry):
        s_wait(ss0)
        s_start(ss0, 2 * j + 2)
        s_wait(ss1)
        s_start(ss1, 2 * j + 3)
        return carry

    lax.fori_loop(0, DEG_CHUNKS // 2 - 1, body, 0)
    s_wait(ss0)
    s_wait(ss1)
    plsc.subcore_barrier()
    _writeback(acc, out_hbm, sid, cid, (r0, r1), (ss0, ss1), B)


_R = 2048
_GRID = NPAD // _R


def _row_mask(i, rows):
    idx = i * rows + lax.broadcasted_iota(jnp.int32, (rows, 1), 0)
    return idx < NN


def _dinv_body(d_ref, o_ref):
    d = d_ref[...]
    deg = d[0, :, 0:1] + d[1, :, 0:1] + 1.0
    dinv = lax.rsqrt(jnp.maximum(deg, 1e-12))
    o_ref[...] = jnp.broadcast_to(dinv, (_R, 128))


def _dinv_tc(d):
    return pl.pallas_call(
        _dinv_body,
        grid=(_GRID,),
        in_specs=[pl.BlockSpec((NC, _R, 128), lambda i: (0, i, 0))],
        out_specs=pl.BlockSpec((_R, 128), lambda i: (i, 0)),
        out_shape=jax.ShapeDtypeStruct((NPAD, 128), _F32),
    )(d)


def _mm1_body(x_ref, w_ref, dv_ref, o_ref):
    i = pl.program_id(0)
    xw = jnp.dot(x_ref[...], w_ref[...], preferred_element_type=_F32,
                 precision=lax.Precision.DEFAULT)
    dv = dv_ref[...][:, 0:1]
    y = jnp.where(_row_mask(i, _R), dv * xw, 0.0)
    o_ref[...] = jnp.stack([y[:, :128], y[:, 128:]], axis=0)


def _mm1_tc(x, W1, dinv):
    return pl.pallas_call(
        _mm1_body,
        grid=(_GRID,),
        in_specs=[
            pl.BlockSpec((_R, 128), lambda i: (i, 0)),
            pl.BlockSpec((128, 256), lambda i: (0, 0)),
            pl.BlockSpec((_R, 128), lambda i: (i, 0)),
        ],
        out_specs=pl.BlockSpec((NC, _R, 128), lambda i: (0, i, 0)),
        out_shape=jax.ShapeDtypeStruct((NC, NPAD, 128), _F32),
    )(x, W1, dinv)


def _mm_mid_body(fout, a_ref, y_ref, dv_ref, b_ref, w_ref, o_ref):
    i = pl.program_id(0)
    s = a_ref[...] + y_ref[...]
    s2 = jnp.concatenate([s[0], s[1]], axis=1)
    dv = dv_ref[...][:, 0:1]
    h = jnp.maximum(dv * s2 + b_ref[...], 0.0)
    xw = jnp.dot(h, w_ref[...], preferred_element_type=_F32,
                 precision=lax.Precision.DEFAULT)
    y = jnp.where(_row_mask(i, _R), dv * xw, 0.0)
    if fout == 256:
        o_ref[...] = jnp.stack([y[:, :128], y[:, 128:]], axis=0)
    else:
        o_ref[...] = jnp.stack([y, y], axis=0)


def _mm_mid_tc(agg, y_prev, dinv, b, W, fout):
    out_shape = jax.ShapeDtypeStruct((NC, NPAD, 128), _F32)
    out_spec = pl.BlockSpec((NC, _R, 128), lambda i: (0, i, 0))
    return pl.pallas_call(
        functools.partial(_mm_mid_body, fout),
        grid=(_GRID,),
        in_specs=[
            pl.BlockSpec((NC, _R, 128), lambda i: (0, i, 0)),
            pl.BlockSpec((NC, _R, 128), lambda i: (0, i, 0)),
            pl.BlockSpec((_R, 128), lambda i: (i, 0)),
            pl.BlockSpec((1, 256), lambda i: (0, 0)),
            pl.BlockSpec((256, fout), lambda i: (0, 0)),
        ],
        out_specs=out_spec,
        out_shape=out_shape,
    )(agg, y_prev, dinv, b, W)


def _final_body(a_ref, y_ref, dv_ref, b_ref, o_ref):
    i = pl.program_id(0)
    a = a_ref[...]
    h = dv_ref[...][:, 0:1] * (a[0] + a[1] + y_ref[...][0]) + b_ref[...]
    h = jnp.where(_row_mask(i, _R), h, jnp.inf)
    m = jnp.min(h, axis=0, keepdims=True)

    @pl.when(i == 0)
    def _():
        o_ref[...] = m

    @pl.when(i > 0)
    def _():
        o_ref[...] = jnp.minimum(o_ref[...], m)


def _final_tc(agg_parts, y3, dinv, b3):
    return pl.pallas_call(
        _final_body,
        grid=(_GRID,),
        in_specs=[
            pl.BlockSpec((NC, _R, 128), lambda i: (0, i, 0)),
            pl.BlockSpec((NC, _R, 128), lambda i: (0, i, 0)),
            pl.BlockSpec((_R, 128), lambda i: (i, 0)),
            pl.BlockSpec((1, 128), lambda i: (0, 0)),
        ],
        out_specs=pl.BlockSpec((1, 128), lambda i: (0, 0)),
        out_shape=jax.ShapeDtypeStruct((1, 128), _F32),
    )(agg_parts, y3, dinv, b3)


def kernel(x, edge_index, W1, b1, W2, b2, W3, b3):
    src = edge_index[0]
    dst = edge_index[1]
    npad_e = EPAD - EE
    src_p = jnp.concatenate([src, jnp.full((npad_e,), PAD_SRC, jnp.int32)])
    dst_p = jnp.concatenate([dst, jnp.full((npad_e,), PAD_DST, jnp.int32)])
    src2 = jnp.concatenate([src_p, src_p + NPAD])
    dst_2d = dst_p.reshape(EPAD // B, B)
    src_2dp = src_p.reshape(EPAD // BP, BP)
    dst_2dp = dst_p.reshape(EPAD // BP, BP)
    src2_2dp = src2.reshape(NC * EPAD // BP, BP)
    xp = jnp.pad(x, ((0, NPAD - NN), (0, 0)))
    b1r = b1.reshape(1, 256)
    b2r = b2.reshape(1, 256)
    b3r = b3.reshape(1, 128)

    deg_parts = _deg_sc(dst_2d).reshape(NC, NPAD, 128)
    dinv = _dinv_tc(deg_parts)
    y1 = _mm1_tc(xp, W1, dinv)
    agg1 = _prop_feature_split(y1.reshape(NC * NPAD, 128), src2_2dp, dst_2dp)
    y2 = _mm_mid_tc(agg1.reshape(NC, NPAD, 128), y1, dinv, b1r, W2, 256)
    agg2 = _prop_feature_split(y2.reshape(NC * NPAD, 128), src2_2dp, dst_2dp)
    y3 = _mm_mid_tc(agg2.reshape(NC, NPAD, 128), y2, dinv, b2r, W3, 128)
    agg3 = _prop_edge_split(y3.reshape(NC * NPAD, 128), src2_2dp, dst_2dp)
    out = _final_tc(agg3.reshape(NC, NPAD, 128), y3, dinv, b3r)
    return out.reshape(128)


# speedup vs baseline: 1.0562x; 1.0004x over previous
import functools

import jax
import jax.numpy as jnp
from jax import lax
from jax.experimental import pallas as pl
from jax.experimental.pallas import tpu as pltpu
from jax.experimental.pallas import tpu_sc as plsc

NN = 10000
EE = 320000
NPAD = 10240
EPAD = 327680
B = 128
NC = 2
NS = 16
ROWS_PER_TILE = NPAD // NS
PAD_SRC = NN
PAD_DST = NN + 16
BP = 64
IDXBUF = 32
NBUF = 4
PROP_CHUNKS_FS = EPAD // (NS * BP)
PROP_CHUNKS_ES = EPAD // (NC * NS * BP)
DEG_CHUNKS = EPAD // (NC * NS * B)

_MESH = plsc.VectorSubcoreMesh(core_axis_name="c", subcore_axis_name="s")
_F32 = jnp.float32


def _fill_rows(buf, nrows, ncols, value):
    vec = jnp.full((16,), value, _F32)

    def body(i, carry):
        for j in range(ncols // 16):
            buf[i, pl.ds(j * 16, 16)] = vec
        return carry

    lax.fori_loop(0, nrows, body, 0)


def _zero_acc_and_sync(r0, acc, sid, nb):
    _fill_rows(r0, nb, 128, 0.0)
    for r in range(ROWS_PER_TILE // nb):
        pltpu.sync_copy(r0, acc.at[pl.ds(sid * ROWS_PER_TILE + r * nb, nb)])


def _writeback(acc, out_hbm, sid, cid, bufs, sems, nb):
    nch = ROWS_PER_TILE // nb
    for r in range(nch):
        row = sid * ROWS_PER_TILE + r * nb
        rb, sem = bufs[r % 2], sems[r % 2]
        if r >= 2:
            prow = cid * NPAD + sid * ROWS_PER_TILE + (r - 2) * nb
            pltpu.make_async_copy(rb, out_hbm.at[pl.ds(prow, nb)], sem).wait()
        pltpu.sync_copy(acc.at[pl.ds(row, nb)], rb)
        pltpu.async_copy(rb, out_hbm.at[pl.ds(cid * NPAD + row, nb)], sem)
    for r in range(max(0, nch - 2), nch):
        row = cid * NPAD + sid * ROWS_PER_TILE + r * nb
        pltpu.make_async_copy(bufs[r % 2], out_hbm.at[pl.ds(row, nb)], sems[r % 2]).wait()


def _make_prop(nchunk, edge_split):
    nstage = nchunk // IDXBUF
    ngroup = IDXBUF // NBUF
    assert nchunk == nstage * IDXBUF and IDXBUF == ngroup * NBUF

    @functools.partial(
        pl.kernel,
        out_type=jax.ShapeDtypeStruct((NC * NPAD, 128), _F32),
        mesh=_MESH,
        scratch_types=[
            pltpu.VMEM((IDXBUF, BP), jnp.int32),
            pltpu.VMEM((IDXBUF, BP), jnp.int32),
            [pltpu.VMEM((BP, 128), _F32)] * NBUF,
            [pltpu.SemaphoreType.DMA] * NBUF,
            [pltpu.SemaphoreType.DMA] * NBUF,
            pltpu.VMEM_SHARED((NPAD, 128), _F32),
        ],
    )
    def prop(y_hbm, srcr_hbm, dstr_hbm, out_hbm,
             sidx, didx, bufs, gsems, ssems, acc):
        cid = lax.axis_index("c")
        sid = lax.axis_index("s")
        if edge_split:
            srow = cid * (EPAD // BP) + (cid * NS + sid) * nchunk
            drow = (cid * NS + sid) * nchunk
        else:
            srow = cid * (EPAD // BP) + sid * nchunk
            drow = sid * nchunk
        _zero_acc_and_sync(bufs[0], acc, sid, BP)
        plsc.subcore_barrier()

        def g_start(b, k):
            pltpu.async_copy(y_hbm.at[sidx.at[k]], bufs[b], gsems[b])

        def g_wait(b):
            pltpu.make_async_copy(y_hbm.at[sidx.at[0]], bufs[b], gsems[b]).wait()

        def s_start(b, k):
            pltpu.async_copy(bufs[b], acc.at[didx.at[k]], ssems[b], add=True)

        def s_wait(b):
            pltpu.make_async_copy(bufs[b], acc.at[didx.at[0]], ssems[b]).wait()

        def body(j, carry):
            for b in range(NBUF):
                k = j * NBUF + b
                g_wait(b)
                s_start(b, k)
                s_wait(b)
                g_start(b, k + NBUF)
            return carry

        for s in range(nstage):
            pltpu.sync_copy(srcr_hbm.at[pl.ds(srow + s * IDXBUF, IDXBUF)], sidx)
            pltpu.sync_copy(dstr_hbm.at[pl.ds(drow + s * IDXBUF, IDXBUF)], didx)
            for b in range(NBUF):
                g_start(b, b)
            lax.fori_loop(0, ngroup - 1, body, 0)
            for b in range(NBUF):
                k = (ngroup - 1) * NBUF + b
                g_wait(b)
                s_start(b, k)
                s_wait(b)
        plsc.subcore_barrier()
        _writeback(acc, out_hbm, sid, cid, (bufs[0], bufs[1]),
                   (gsems[0], gsems[1]), BP)

    return prop


_prop_feature_split = _make_prop(PROP_CHUNKS_FS, edge_split=False)
_prop_edge_split = _make_prop(PROP_CHUNKS_ES, edge_split=True)


@functools.partial(
    pl.kernel,
    out_type=jax.ShapeDtypeStruct((NC * NPAD, 128), _F32),
    mesh=_MESH,
    scratch_types=[
        pltpu.VMEM((DEG_CHUNKS, B), jnp.int32),
        pltpu.VMEM((B, 128), _F32),
        pltpu.VMEM((B, 128), _F32),
        pltpu.SemaphoreType.DMA,
        pltpu.SemaphoreType.DMA,
        pltpu.VMEM_SHARED((NPAD, 128), _F32),
    ],
)
def _deg_sc(dstr_hbm, out_hbm, didx, r0, r1, ss0, ss1, acc):
    cid = lax.axis_index("c")
    sid = lax.axis_index("s")
    drow = (cid * NS + sid) * DEG_CHUNKS
    pltpu.sync_copy(dstr_hbm.at[pl.ds(drow, DEG_CHUNKS)], didx)
    _zero_acc_and_sync(r0, acc, sid, B)
    _fill_rows(r1, B, 128, 1.0)
    plsc.subcore_barrier()

    def s_start(sem, k):
        pltpu.async_copy(r1, acc.at[didx.at[k]], sem, add=True)

    def s_wait(sem):
        pltpu.make_async_copy(r1, acc.at[didx.at[0]], sem).wait()

    s_start(ss0, 0)
    s_start(ss1, 1)

    def body(j, carry):
        s_wait(ss0)
        s_start(ss0, 2 * j + 2)
        s_wait(ss1)
        s_start(ss1, 2 * j + 3)
        return carry

    lax.fori_loop(0, DEG_CHUNKS // 2 - 1, body, 0)
    s_wait(ss0)
    s_wait(ss1)
    plsc.subcore_barrier()
    _writeback(acc, out_hbm, sid, cid, (r0, r1), (ss0, ss1), B)


_R = 2048
_GRID = NPAD // _R


def _row_mask(i, rows):
    idx = i * rows + lax.broadcasted_iota(jnp.int32, (rows, 1), 0)
    return idx < NN


def _dinv_body(d_ref, o_ref):
    d = d_ref[...]
    deg = d[0, :, 0:1] + d[1, :, 0:1] + 1.0
    dinv = lax.rsqrt(jnp.maximum(deg, 1e-12))
    o_ref[...] = jnp.broadcast_to(dinv, (_R, 128))


def _dinv_tc(d):
    return pl.pallas_call(
        _dinv_body,
        grid=(_GRID,),
        in_specs=[pl.BlockSpec((NC, _R, 128), lambda i: (0, i, 0))],
        out_specs=pl.BlockSpec((_R, 128), lambda i: (i, 0)),
        out_shape=jax.ShapeDtypeStruct((NPAD, 128), _F32),
    )(d)


def _mm1_body(x_ref, w_ref, dv_ref, o_ref):
    i = pl.program_id(0)
    xw = jnp.dot(x_ref[...], w_ref[...], preferred_element_type=_F32,
                 precision=lax.Precision.DEFAULT)
    dv = dv_ref[...][:, 0:1]
    y = jnp.where(_row_mask(i, _R), dv * xw, 0.0)
    o_ref[...] = jnp.stack([y[:, :128], y[:, 128:]], axis=0)


def _mm1_tc(x, W1, dinv):
    return pl.pallas_call(
        _mm1_body,
        grid=(_GRID,),
        in_specs=[
            pl.BlockSpec((_R, 128), lambda i: (i, 0)),
            pl.BlockSpec((128, 256), lambda i: (0, 0)),
            pl.BlockSpec((_R, 128), lambda i: (i, 0)),
        ],
        out_specs=pl.BlockSpec((NC, _R, 128), lambda i: (0, i, 0)),
        out_shape=jax.ShapeDtypeStruct((NC, NPAD, 128), _F32),
    )(x, W1, dinv)


def _mm_mid_body(fout, a_ref, y_ref, dv_ref, b_ref, w_ref, o_ref):
    i = pl.program_id(0)
    s = a_ref[...] + y_ref[...]
    s2 = jnp.concatenate([s[0], s[1]], axis=1)
    dv = dv_ref[...][:, 0:1]
    h = jnp.maximum(dv * s2 + b_ref[...], 0.0)
    xw = jnp.dot(h, w_ref[...], preferred_element_type=_F32,
                 precision=lax.Precision.DEFAULT)
    y = jnp.where(_row_mask(i, _R), dv * xw, 0.0)
    if fout == 256:
        o_ref[...] = jnp.stack([y[:, :128], y[:, 128:]], axis=0)
    else:
        o_ref[...] = jnp.stack([y, y], axis=0)


def _mm_mid_tc(agg, y_prev, dinv, b, W, fout):
    out_shape = jax.ShapeDtypeStruct((NC, NPAD, 128), _F32)
    out_spec = pl.BlockSpec((NC, _R, 128), lambda i: (0, i, 0))
    return pl.pallas_call(
        functools.partial(_mm_mid_body, fout),
        grid=(_GRID,),
        in_specs=[
            pl.BlockSpec((NC, _R, 128), lambda i: (0, i, 0)),
            pl.BlockSpec((NC, _R, 128), lambda i: (0, i, 0)),
            pl.BlockSpec((_R, 128), lambda i: (i, 0)),
            pl.BlockSpec((1, 256), lambda i: (0, 0)),
            pl.BlockSpec((256, fout), lambda i: (0, 0)),
        ],
        out_specs=out_spec,
        out_shape=out_shape,
    )(agg, y_prev, dinv, b, W)


def _final_body(a_ref, y_ref, dv_ref, b_ref, o_ref):
    i = pl.program_id(0)
    a = a_ref[...]
    h = dv_ref[...][:, 0:1] * (a[0] + a[1] + y_ref[...][0]) + b_ref[...]
    h = jnp.where(_row_mask(i, _R), h, jnp.inf)
    m = jnp.min(h, axis=0, keepdims=True)

    @pl.when(i == 0)
    def _():
        o_ref[...] = m

    @pl.when(i > 0)
    def _():
        o_ref[...] = jnp.minimum(o_ref[...], m)


def _final_tc(agg_parts, y3, dinv, b3):
    return pl.pallas_call(
        _final_body,
        grid=(_GRID,),
        in_specs=[
            pl.BlockSpec((NC, _R, 128), lambda i: (0, i, 0)),
            pl.BlockSpec((NC, _R, 128), lambda i: (0, i, 0)),
            pl.BlockSpec((_R, 128), lambda i: (i, 0)),
            pl.BlockSpec((1, 128), lambda i: (0, 0)),
        ],
        out_specs=pl.BlockSpec((1, 128), lambda i: (0, 0)),
        out_shape=jax.ShapeDtypeStruct((1, 128), _F32),
    )(agg_parts, y3, dinv, b3)


def kernel(x, edge_index, W1, b1, W2, b2, W3, b3):
    src = edge_index[0]
    dst = edge_index[1]
    npad_e = EPAD - EE
    src_p = jnp.concatenate([src, jnp.full((npad_e,), PAD_SRC, jnp.int32)])
    dst_p = jnp.concatenate([dst, jnp.full((npad_e,), PAD_DST, jnp.int32)])
    src2 = jnp.concatenate([src_p, src_p + NPAD])
    dst_2d = dst_p.reshape(EPAD // B, B)
    dst_2dp = dst_p.reshape(EPAD // BP, BP)
    src2_2dp = src2.reshape(NC * EPAD // BP, BP)
    xp = jnp.pad(x, ((0, NPAD - NN), (0, 0)))
    b1r = b1.reshape(1, 256)
    b2r = b2.reshape(1, 256)
    b3r = b3.reshape(1, 128)

    deg_parts = _deg_sc(dst_2d).reshape(NC, NPAD, 128)
    dinv = _dinv_tc(deg_parts)
    y1 = _mm1_tc(xp, W1, dinv)
    agg1 = _prop_feature_split(y1.reshape(NC * NPAD, 128), src2_2dp, dst_2dp)
    y2 = _mm_mid_tc(agg1.reshape(NC, NPAD, 128), y1, dinv, b1r, W2, 256)
    agg2 = _prop_feature_split(y2.reshape(NC * NPAD, 128), src2_2dp, dst_2dp)
    y3 = _mm_mid_tc(agg2.reshape(NC, NPAD, 128), y2, dinv, b2r, W3, 128)
    agg3 = _prop_edge_split(y3.reshape(NC * NPAD, 128), src2_2dp, dst_2dp)
    out = _final_tc(agg3.reshape(NC, NPAD, 128), y3, dinv, b3r)
    return out.reshape(128)
